# baseline (speedup 1.0000x reference)
import functools
import jax
import jax.numpy as jnp
from jax import lax
from jax.experimental import pallas as pl
from jax.experimental.pallas import tpu as pltpu
from jax.experimental.pallas import tpu_sc as plsc

_B, _T, _N, _D = 8, 16, 196, 768
_FRAME = _N * _D
_NPIECE = 16
_PIECE = _FRAME // _NPIECE
_NJOB = (_B * _T) // 2
_VECS = _PIECE // 16


def _sc_body(x_hbm, tbl_hbm, out_hbm,
             tbl_v, in0, in1, out0, out1,
             sin0, sin1, sout0, sout1):
    wid = lax.axis_index("c") * 16 + lax.axis_index("s")
    piece = wid // 2
    half = wid % 2
    tbl_off = piece * _PIECE
    frame0 = half * _NJOB

    pltpu.sync_copy(tbl_hbm.at[pl.ds(tbl_off, _PIECE)], tbl_v)

    ins = (in0, in1)
    outs = (out0, out1)
    sins = (sin0, sin1)
    souts = (sout0, sout1)

    def x_slice(j):
        return pl.ds((frame0 + j) * _FRAME + tbl_off, _PIECE)

    pltpu.async_copy(x_hbm.at[x_slice(0)], in0, sin0)
    pltpu.async_copy(x_hbm.at[x_slice(1)], in1, sin1)

    @pl.loop(0, _NJOB, step=2)
    def jobs(jbase):
        for b in range(2):
            j = jbase + b
            i_v, o_v, s_in, s_out = ins[b], outs[b], sins[b], souts[b]
            pltpu.make_async_copy(x_hbm.at[x_slice(j)], i_v, s_in).wait()

            @pl.when(j >= 2)
            def _():
                pltpu.make_async_copy(o_v, out_hbm.at[x_slice(j)], s_out).wait()

            @pl.loop(0, _VECS, unroll=8)
            def add(i):
                off = i * 16
                o_v[pl.ds(off, 16)] = i_v[pl.ds(off, 16)] + tbl_v[pl.ds(off, 16)]

            pltpu.async_copy(o_v, out_hbm.at[x_slice(j)], s_out)

            @pl.when(j + 2 < _NJOB)
            def _():
                pltpu.async_copy(x_hbm.at[x_slice(j + 2)], i_v, s_in)

    pltpu.make_async_copy(out0, out_hbm.at[x_slice(_NJOB - 2)], sout0).wait()
    pltpu.make_async_copy(out1, out_hbm.at[x_slice(_NJOB - 1)], sout1).wait()


def kernel(encoded_tokens, pos_table):
    B, T, N, D = encoded_tokens.shape
    x = encoded_tokens.reshape(B * T * N * D)
    tbl = pos_table.reshape(N * D)
    mesh = plsc.VectorSubcoreMesh(core_axis_name="c", subcore_axis_name="s")
    run = pl.kernel(
        _sc_body,
        mesh=mesh,
        out_type=jax.ShapeDtypeStruct((B * T * N * D,), jnp.float32),
        scratch_types=[
            pltpu.VMEM((_PIECE,), jnp.float32),
            pltpu.VMEM((_PIECE,), jnp.float32),
            pltpu.VMEM((_PIECE,), jnp.float32),
            pltpu.VMEM((_PIECE,), jnp.float32),
            pltpu.VMEM((_PIECE,), jnp.float32),
            pltpu.SemaphoreType.DMA,
            pltpu.SemaphoreType.DMA,
            pltpu.SemaphoreType.DMA,
            pltpu.SemaphoreType.DMA,
        ],
    )
    out = run(x, tbl)
    return out.reshape(B, T, N, D)


# speedup vs baseline: 1.3720x; 1.3720x over previous
import functools
import jax
import jax.numpy as jnp
from jax import lax
from jax.experimental import pallas as pl
from jax.experimental.pallas import tpu as pltpu
from jax.experimental.pallas import tpu_sc as plsc

_B, _T, _N, _D = 8, 16, 196, 768
_FRAME = _N * _D
_NPIECE = 16
_PIECE = _FRAME // _NPIECE
_NJOB = (_B * _T) // 2
_VECS = _PIECE // 16


def _sc_body(x_hbm, tbl_hbm, out_hbm,
             tbl_v, in0, in1, out0, out1,
             sin0, sin1, sout0, sout1):
    wid = lax.axis_index("c") * 16 + lax.axis_index("s")
    piece = wid // 2
    half = wid % 2
    tbl_off = piece * _PIECE
    frame0 = half * _NJOB

    pltpu.sync_copy(tbl_hbm.at[pl.ds(tbl_off, _PIECE)], tbl_v)

    ins = (in0, in1)
    outs = (out0, out1)
    sins = (sin0, sin1)
    souts = (sout0, sout1)

    def x_slice(j):
        return pl.ds((frame0 + j) * _FRAME + tbl_off, _PIECE)

    pltpu.async_copy(x_hbm.at[x_slice(0)], in0, sin0)
    pltpu.async_copy(x_hbm.at[x_slice(1)], in1, sin1)

    @pl.loop(0, _NJOB, step=2)
    def jobs(jbase):
        for b in range(2):
            j = jbase + b
            i_v, o_v, s_in, s_out = ins[b], outs[b], sins[b], souts[b]
            pltpu.make_async_copy(x_hbm.at[x_slice(j)], i_v, s_in).wait()

            @pl.when(j >= 2)
            def _():
                pltpu.make_async_copy(o_v, out_hbm.at[x_slice(j)], s_out).wait()

            @plsc.parallel_loop(0, _VECS, 1, unroll=8)
            def add(i):
                off = i * 16
                o_v[pl.ds(off, 16)] = i_v[pl.ds(off, 16)] + tbl_v[pl.ds(off, 16)]

            pltpu.async_copy(o_v, out_hbm.at[x_slice(j)], s_out)

            @pl.when(j + 2 < _NJOB)
            def _():
                pltpu.async_copy(x_hbm.at[x_slice(j + 2)], i_v, s_in)

    pltpu.make_async_copy(out0, out_hbm.at[x_slice(_NJOB - 2)], sout0).wait()
    pltpu.make_async_copy(out1, out_hbm.at[x_slice(_NJOB - 1)], sout1).wait()


def kernel(encoded_tokens, pos_table):
    B, T, N, D = encoded_tokens.shape
    x = encoded_tokens.reshape(B * T * N * D)
    tbl = pos_table.reshape(N * D)
    mesh = plsc.VectorSubcoreMesh(core_axis_name="c", subcore_axis_name="s")
    run = pl.kernel(
        _sc_body,
        mesh=mesh,
        out_type=jax.ShapeDtypeStruct((B * T * N * D,), jnp.float32),
        scratch_types=[
            pltpu.VMEM((_PIECE,), jnp.float32),
            pltpu.VMEM((_PIECE,), jnp.float32),
            pltpu.VMEM((_PIECE,), jnp.float32),
            pltpu.VMEM((_PIECE,), jnp.float32),
            pltpu.VMEM((_PIECE,), jnp.float32),
            pltpu.SemaphoreType.DMA,
            pltpu.SemaphoreType.DMA,
            pltpu.SemaphoreType.DMA,
            pltpu.SemaphoreType.DMA,
        ],
    )
    out = run(x, tbl)
    return out.reshape(B, T, N, D)


# speedup vs baseline: 1.4356x; 1.0464x over previous
import jax
import jax.numpy as jnp
from jax import lax
from jax.experimental import pallas as pl
from jax.experimental.pallas import tpu as pltpu
from jax.experimental.pallas import tpu_sc as plsc

_B, _T, _N, _D = 8, 16, 196, 768
_FRAME = _N * _D
_NPIECE = 16
_PIECE = _FRAME // _NPIECE
_NJOB = (_B * _T) // 2
_VECS = _PIECE // 16
_NBUF = 4


def _sc_body(x_hbm, tbl_hbm, out_hbm, tbl_v, ins, outs, sins, souts):
    wid = lax.axis_index("c") * 16 + lax.axis_index("s")
    piece = wid // 2
    half = wid % 2
    tbl_off = piece * _PIECE
    frame0 = half * _NJOB

    pltpu.sync_copy(tbl_hbm.at[pl.ds(tbl_off, _PIECE)], tbl_v)

    def x_slice(j):
        return pl.ds((frame0 + j) * _FRAME + tbl_off, _PIECE)

    for b in range(_NBUF):
        pltpu.async_copy(x_hbm.at[x_slice(b)], ins[b], sins[b])

    @pl.loop(0, _NJOB, step=_NBUF)
    def jobs(jbase):
        for b in range(_NBUF):
            j = jbase + b
            i_v, o_v, s_in, s_out = ins[b], outs[b], sins[b], souts[b]
            pltpu.make_async_copy(x_hbm.at[x_slice(j)], i_v, s_in).wait()

            @pl.when(j >= _NBUF)
            def _():
                pltpu.make_async_copy(o_v, out_hbm.at[x_slice(j)], s_out).wait()

            @plsc.parallel_loop(0, _VECS, 1, unroll=8)
            def add(i):
                off = i * 16
                o_v[pl.ds(off, 16)] = i_v[pl.ds(off, 16)] + tbl_v[pl.ds(off, 16)]

            pltpu.async_copy(o_v, out_hbm.at[x_slice(j)], s_out)

            @pl.when(j + _NBUF < _NJOB)
            def _():
                pltpu.async_copy(x_hbm.at[x_slice(j + _NBUF)], i_v, s_in)

    for b in range(_NBUF):
        pltpu.make_async_copy(
            outs[b], out_hbm.at[x_slice(_NJOB - _NBUF + b)], souts[b]
        ).wait()


def kernel(encoded_tokens, pos_table):
    B, T, N, D = encoded_tokens.shape
    x = encoded_tokens.reshape(B * T * N * D)
    tbl = pos_table.reshape(N * D)
    mesh = plsc.VectorSubcoreMesh(core_axis_name="c", subcore_axis_name="s")
    run = pl.kernel(
        _sc_body,
        mesh=mesh,
        out_type=jax.ShapeDtypeStruct((B * T * N * D,), jnp.float32),
        scratch_types=[
            pltpu.VMEM((_PIECE,), jnp.float32),
            [pltpu.VMEM((_PIECE,), jnp.float32) for _ in range(_NBUF)],
            [pltpu.VMEM((_PIECE,), jnp.float32) for _ in range(_NBUF)],
            [pltpu.SemaphoreType.DMA for _ in range(_NBUF)],
            [pltpu.SemaphoreType.DMA for _ in range(_NBUF)],
        ],
    )
    out = run(x, tbl)
    return out.reshape(B, T, N, D)


# speedup vs baseline: 2.0001x; 1.3932x over previous
import jax
import jax.numpy as jnp
from jax import lax
from jax.experimental import pallas as pl
from jax.experimental.pallas import tpu as pltpu
from jax.experimental.pallas import tpu_sc as plsc

_B, _T, _N, _D = 8, 16, 196, 768
_ROWS = 32
_NFULL = _N // _ROWS
_TAIL = _N - _NFULL * _ROWS
_SLABS = (_B * _T) // 32
_RVECS = _D // 16


def _sc_body(x_hbm, tbl_hbm, out_hbm, tbl_v, ins, outs, sins, souts):
    wid = lax.axis_index("c") * 16 + lax.axis_index("s")
    slab0 = wid * _SLABS

    def bt(f):
        s = slab0 + f
        return s // _T, lax.rem(s, _T)

    def chunk_pass(row0, nrows):
        rows = pl.ds(row0, nrows)
        pltpu.sync_copy(tbl_hbm.at[rows, :], tbl_v.at[pl.ds(0, nrows), :])

        for b in range(2):
            bb, tb = bt(b)
            pltpu.async_copy(
                x_hbm.at[bb, tb, rows, :], ins[b].at[pl.ds(0, nrows), :], sins[b]
            )

        @pl.loop(0, _SLABS, step=2)
        def jobs(jbase):
            for b in range(2):
                j = jbase + b
                i_v, o_v, s_in, s_out = ins[b], outs[b], sins[b], souts[b]
                bj, tj = bt(j)
                pltpu.make_async_copy(
                    x_hbm.at[bj, tj, rows, :], i_v.at[pl.ds(0, nrows), :], s_in
                ).wait()

                @pl.when(j >= 2)
                def _():
                    pltpu.make_async_copy(
                        o_v.at[pl.ds(0, nrows), :],
                        out_hbm.at[bj, tj, rows, :],
                        s_out,
                    ).wait()

                @pl.loop(0, nrows)
                def row_add(r):
                    @plsc.parallel_loop(0, _RVECS, 1, unroll=8)
                    def add(i):
                        off = i * 16
                        o_v[r, pl.ds(off, 16)] = (
                            i_v[r, pl.ds(off, 16)] + tbl_v[r, pl.ds(off, 16)]
                        )

                pltpu.async_copy(
                    o_v.at[pl.ds(0, nrows), :], out_hbm.at[bj, tj, rows, :], s_out
                )

                @pl.when(j + 2 < _SLABS)
                def _():
                    bn, tn = bt(j + 2)
                    pltpu.async_copy(
                        x_hbm.at[bn, tn, rows, :],
                        i_v.at[pl.ds(0, nrows), :],
                        s_in,
                    )

        for b in range(2):
            bb, tb = bt(_SLABS - 2 + b)
            pltpu.make_async_copy(
                outs[b].at[pl.ds(0, nrows), :],
                out_hbm.at[bb, tb, rows, :],
                souts[b],
            ).wait()

    for c in range(_NFULL):
        chunk_pass(c * _ROWS, _ROWS)
    chunk_pass(_NFULL * _ROWS, _TAIL)


def kernel(encoded_tokens, pos_table):
    B, T, N, D = encoded_tokens.shape
    mesh = plsc.VectorSubcoreMesh(core_axis_name="c", subcore_axis_name="s")
    run = pl.kernel(
        _sc_body,
        mesh=mesh,
        out_type=jax.ShapeDtypeStruct((B, T, N, D), jnp.float32),
        scratch_types=[
            pltpu.VMEM((_ROWS, _D), jnp.float32),
            [pltpu.VMEM((_ROWS, _D), jnp.float32) for _ in range(2)],
            [pltpu.VMEM((_ROWS, _D), jnp.float32) for _ in range(2)],
            [pltpu.SemaphoreType.DMA for _ in range(2)],
            [pltpu.SemaphoreType.DMA for _ in range(2)],
        ],
    )
    return run(encoded_tokens, pos_table)


# speedup vs baseline: 5.4381x; 2.7189x over previous
import jax
import jax.numpy as jnp
from jax import lax
from jax.experimental import pallas as pl
from jax.experimental.pallas import tpu as pltpu
from jax.experimental.pallas import tpu_sc as plsc

_B, _T, _N, _D = 8, 16, 196, 768
_K = 2
_RVECS = _D // 16
_PROWS = 24
_NJOB = _PROWS // _K
_NTAIL = _N - 4 * 48


def _sc_body(x_hbm, tbl_hbm, out_hbm, tbl_v, ins, outs, sins, souts):
    wid = lax.axis_index("c") * 16 + lax.axis_index("s")
    b = wid // 4
    g = lax.rem(wid, 4)

    def do_add(i_v, o_v, row):
        @pl.loop(0, _T)
        def t_add(t):
            @plsc.parallel_loop(0, _RVECS, 1, unroll=8)
            def add(i):
                off = i * 16
                o_v[t, pl.ds(off, 16)] = (
                    i_v[t, pl.ds(off, 16)] + tbl_v[row, pl.ds(off, 16)]
                )

    def phase(n0):
        pltpu.sync_copy(
            tbl_hbm.at[pl.ds(n0, _PROWS), :], tbl_v.at[pl.ds(0, _PROWS), :]
        )

        def x_at(j):
            return pl.ds(n0 + j * _K, _K)

        for r in range(2):
            pltpu.async_copy(x_hbm.at[b, x_at(r), :, :], ins[r], sins[r])

        @pl.loop(0, _NJOB, step=2)
        def jobs(jbase):
            for r in range(2):
                j = jbase + r
                i_v, o_v, s_in, s_out = ins[r], outs[r], sins[r], souts[r]
                pltpu.make_async_copy(
                    x_hbm.at[b, x_at(j), :, :], i_v, s_in
                ).wait()

                @pl.when(j >= 2)
                def _():
                    pltpu.make_async_copy(
                        o_v, out_hbm.at[b, x_at(j), :, :], s_out
                    ).wait()

                for s in range(_K):
                    do_add(i_v.at[s], o_v.at[s], j * _K + s)

                pltpu.async_copy(o_v, out_hbm.at[b, x_at(j), :, :], s_out)

                @pl.when(j + 2 < _NJOB)
                def _():
                    pltpu.async_copy(x_hbm.at[b, x_at(j + 2), :, :], i_v, s_in)

        for r in range(2):
            pltpu.make_async_copy(
                outs[r], out_hbm.at[b, x_at(_NJOB - 2 + r), :, :], souts[r]
            ).wait()

    phase(48 * g)
    phase(48 * g + _PROWS)

    pltpu.sync_copy(tbl_hbm.at[pl.ds(192, _NTAIL), :], tbl_v.at[pl.ds(0, _NTAIL), :])
    n_tail = 192 + g
    pltpu.sync_copy(x_hbm.at[b, n_tail], ins[0].at[0])
    do_add(ins[0].at[0], outs[0].at[0], g)
    pltpu.sync_copy(outs[0].at[0], out_hbm.at[b, n_tail])


def kernel(encoded_tokens, pos_table):
    B, T, N, D = encoded_tokens.shape
    xt = jnp.transpose(encoded_tokens, (0, 2, 1, 3))
    mesh = plsc.VectorSubcoreMesh(core_axis_name="c", subcore_axis_name="s")
    run = pl.kernel(
        _sc_body,
        mesh=mesh,
        out_type=jax.ShapeDtypeStruct((B, N, T, D), jnp.float32),
        scratch_types=[
            pltpu.VMEM((_PROWS, _D), jnp.float32),
            [pltpu.VMEM((_K, _T, _D), jnp.float32) for _ in range(2)],
            [pltpu.VMEM((_K, _T, _D), jnp.float32) for _ in range(2)],
            [pltpu.SemaphoreType.DMA for _ in range(2)],
            [pltpu.SemaphoreType.DMA for _ in range(2)],
        ],
    )
    out_t = run(xt, pos_table)
    return jnp.transpose(out_t, (0, 2, 1, 3))


# speedup vs baseline: 5.8378x; 1.0735x over previous
import jax
import jax.numpy as jnp
from jax import lax
from jax.experimental import pallas as pl
from jax.experimental.pallas import tpu as pltpu
from jax.experimental.pallas import tpu_sc as plsc

_B, _T, _N, _D = 8, 16, 196, 768
_RVECS = _D // 16
_GROWS = 48
_NTAIL = _N - 4 * _GROWS
_NB = 6
_LEAD = 3


def _sc_body(x_hbm, tbl_hbm, out_hbm, tbl_v, bufs, sins, souts):
    wid = lax.axis_index("c") * 16 + lax.axis_index("s")
    b = wid // 4
    g = lax.rem(wid, 4)
    n0 = _GROWS * g

    def add_row(v, row):
        @pl.loop(0, _T)
        def t_add(t):
            @plsc.parallel_loop(0, _RVECS, 1, unroll=8)
            def add(i):
                off = i * 16
                plsc.addupdate(
                    v.at[t, pl.ds(off, 16)], tbl_v[row, pl.ds(off, 16)]
                )

    pltpu.sync_copy(tbl_hbm.at[pl.ds(n0, _GROWS), :], tbl_v)

    for r in range(_LEAD):
        pltpu.async_copy(x_hbm.at[b, n0 + r], bufs[r], sins[r])

    @pl.loop(0, _GROWS, step=_NB)
    def jobs(jbase):
        for r in range(_NB):
            j = jbase + r
            v, s_in, s_out = bufs[r], sins[r], souts[r]
            pltpu.make_async_copy(x_hbm.at[b, n0 + j], v, s_in).wait()
            add_row(v, j)
            pltpu.async_copy(v, out_hbm.at[b, n0 + j], s_out)

            jn = j + _LEAD
            rn = (r + _LEAD) % _NB

            @pl.when(jn < _GROWS)
            def _():
                @pl.when(jn >= _NB)
                def _():
                    pltpu.make_async_copy(
                        bufs[rn], out_hbm.at[b, n0 + jn - _NB], souts[rn]
                    ).wait()

                pltpu.async_copy(x_hbm.at[b, n0 + jn], bufs[rn], sins[rn])

    for r in range(_NB):
        j = _GROWS - _NB + r
        pltpu.make_async_copy(
            bufs[r], out_hbm.at[b, n0 + j], souts[r]
        ).wait()

    pltpu.sync_copy(
        tbl_hbm.at[pl.ds(4 * _GROWS, _NTAIL), :], tbl_v.at[pl.ds(0, _NTAIL), :]
    )
    n_tail = 4 * _GROWS + g
    pltpu.sync_copy(x_hbm.at[b, n_tail], bufs[0])
    add_row(bufs[0], g)
    pltpu.sync_copy(bufs[0], out_hbm.at[b, n_tail])


def kernel(encoded_tokens, pos_table):
    B, T, N, D = encoded_tokens.shape
    xt = jnp.transpose(encoded_tokens, (0, 2, 1, 3))
    mesh = plsc.VectorSubcoreMesh(core_axis_name="c", subcore_axis_name="s")
    run = pl.kernel(
        _sc_body,
        mesh=mesh,
        out_type=jax.ShapeDtypeStruct((B, N, T, D), jnp.float32),
        scratch_types=[
            pltpu.VMEM((_GROWS, _D), jnp.float32),
            [pltpu.VMEM((_T, _D), jnp.float32) for _ in range(_NB)],
            [pltpu.SemaphoreType.DMA for _ in range(_NB)],
            [pltpu.SemaphoreType.DMA for _ in range(_NB)],
        ],
    )
    out_t = run(xt, pos_table)
    return jnp.transpose(out_t, (0, 2, 1, 3))


# speedup vs baseline: 5.9353x; 1.0167x over previous
import jax
import jax.numpy as jnp
from jax import lax
from jax.experimental import pallas as pl
from jax.experimental.pallas import tpu as pltpu
from jax.experimental.pallas import tpu_sc as plsc

_B, _T, _N, _D = 8, 16, 196, 768
_RVECS = _D // 16
_GROWS = 48
_NTAIL = _N - 4 * _GROWS
_NB = 6
_LEAD = 3


def _sc_body(x_hbm, tbl_hbm, out_hbm, tbl_v, bufs, sins, souts):
    wid = lax.axis_index("c") * 16 + lax.axis_index("s")
    b = wid // 4
    g = lax.rem(wid, 4)
    n0 = _GROWS * g

    def add_row(v, row):
        @plsc.parallel_loop(0, _RVECS, 1, unroll=2)
        def add(i):
            tv = tbl_v[row, pl.ds(i * 16, 16)]
            for t in range(_T):
                plsc.addupdate(v.at[t, pl.ds(i * 16, 16)], tv)

    pltpu.sync_copy(tbl_hbm.at[pl.ds(n0, _GROWS), :], tbl_v)

    for r in range(_LEAD):
        pltpu.async_copy(x_hbm.at[b, n0 + r], bufs[r], sins[r])

    @pl.loop(0, _GROWS, step=_NB)
    def jobs(jbase):
        for r in range(_NB):
            j = jbase + r
            v, s_in, s_out = bufs[r], sins[r], souts[r]
            pltpu.make_async_copy(x_hbm.at[b, n0 + j], v, s_in).wait()
            add_row(v, j)
            pltpu.async_copy(v, out_hbm.at[b, n0 + j], s_out)

            jn = j + _LEAD
            rn = (r + _LEAD) % _NB

            @pl.when(jn < _GROWS)
            def _():
                @pl.when(jn >= _NB)
                def _():
                    pltpu.make_async_copy(
                        bufs[rn], out_hbm.at[b, n0 + jn - _NB], souts[rn]
                    ).wait()

                pltpu.async_copy(x_hbm.at[b, n0 + jn], bufs[rn], sins[rn])

    for r in range(_NB):
        j = _GROWS - _NB + r
        pltpu.make_async_copy(
            bufs[r], out_hbm.at[b, n0 + j], souts[r]
        ).wait()

    pltpu.sync_copy(
        tbl_hbm.at[pl.ds(4 * _GROWS, _NTAIL), :], tbl_v.at[pl.ds(0, _NTAIL), :]
    )
    n_tail = 4 * _GROWS + g
    pltpu.sync_copy(x_hbm.at[b, n_tail], bufs[0])
    add_row(bufs[0], g)
    pltpu.sync_copy(bufs[0], out_hbm.at[b, n_tail])


def kernel(encoded_tokens, pos_table):
    B, T, N, D = encoded_tokens.shape
    xt = jnp.transpose(encoded_tokens, (0, 2, 1, 3))
    mesh = plsc.VectorSubcoreMesh(core_axis_name="c", subcore_axis_name="s")
    run = pl.kernel(
        _sc_body,
        mesh=mesh,
        out_type=jax.ShapeDtypeStruct((B, N, T, D), jnp.float32),
        scratch_types=[
            pltpu.VMEM((_GROWS, _D), jnp.float32),
            [pltpu.VMEM((_T, _D), jnp.float32) for _ in range(_NB)],
            [pltpu.SemaphoreType.DMA for _ in range(_NB)],
            [pltpu.SemaphoreType.DMA for _ in range(_NB)],
        ],
    )
    out_t = run(xt, pos_table)
    return jnp.transpose(out_t, (0, 2, 1, 3))
